# baseline (speedup 1.0000x reference)
import functools

import jax
import jax.numpy as jnp
from jax.experimental import pallas as pl
from jax.experimental.pallas import tpu as pltpu
from jax.experimental.pallas import tpu_sc as plsc

_K = 5
_P = 3
_S = 1

_Q = 2116
_D = 2304
_QB = 128
_NBLK = 17
_QPAD = _QB * _NBLK
_NEG = -3.0e38


def _extract_patches_t(feat, p, s):
    xs = feat[0]
    nh = xs.shape[1] - p + 1
    win = jnp.stack([xs[:, dy:dy + nh, dx:dx + nh]
                     for dy in range(p) for dx in range(p)], axis=1)
    return win.reshape(xs.shape[0] * p * p, nh * nh)


def _pin_body(n_ref, o_ref):
    o_ref[...] = n_ref[...]


def _transpose_body(qt_ref, kt_ref, q_ref, k_ref):
    q_ref[...] = qt_ref[...].T
    k_ref[...] = kt_ref[...].T


def _rows_from_t(qt, kt):
    return pl.pallas_call(
        _transpose_body,
        grid=(_NBLK,),
        in_specs=[
            pl.BlockSpec((_D, _QB), lambda i: (0, i)),
            pl.BlockSpec((_D, _QB), lambda i: (0, i)),
        ],
        out_specs=[
            pl.BlockSpec((_QB, _D), lambda i: (i, 0)),
            pl.BlockSpec((_QB, _D), lambda i: (i, 0)),
        ],
        out_shape=[
            jax.ShapeDtypeStruct((_Q, _D), jnp.float32),
            jax.ShapeDtypeStruct((_Q, _D), jnp.float32),
        ],
    )(qt, kt)


def _topk_body(q_ref, kt_ref, sc_ref, ix_ref, kb16_ref, k2_ref):
    i = pl.program_id(0)

    @pl.when(i == 0)
    def _():
        kt = kt_ref[...]
        kb16_ref[...] = kt.astype(jnp.bfloat16)
        k2_ref[0:1, :] = jnp.sum(kt * kt, axis=0, keepdims=True)

    qb = q_ref[...]
    q2 = jnp.sum(qb * qb, axis=1)
    qk = jax.lax.dot_general(
        qb.astype(jnp.bfloat16), kb16_ref[...],
        dimension_numbers=(((1,), (0,)), ((), ())),
        preferred_element_type=jnp.float32)
    t = 2.0 * qk - k2_ref[0:1, :]
    col = jax.lax.broadcasted_iota(jnp.int32, (_QB, _Q), 1)
    for kk in range(_K):
        m = jnp.max(t, axis=1)
        a = jnp.argmax(t, axis=1).astype(jnp.int32)
        sc_ref[:, kk] = m - q2
        ix_ref[:, kk] = a
        t = jnp.where(col == a[:, None], _NEG, t)


_NW = 32
_TPW = 352
_TPAD = _NW * _TPW
_W = 8
_NWIN = _TPW // _W
_NBUF = 4


def _sc_gather(kpp, idx2d, tidx2d):
    mesh = plsc.VectorSubcoreMesh(core_axis_name="c", subcore_axis_name="s")

    @functools.partial(
        pl.kernel, mesh=mesh,
        out_type=jax.ShapeDtypeStruct((_QPAD * 8, _D), jnp.float32),
        scratch_types=(
            [pltpu.VMEM((_NWIN, _W), jnp.int32),
             pltpu.VMEM((_NWIN, _W), jnp.int32)]
            + [pltpu.VMEM((_W, _D), jnp.float32)] * _NBUF
            + [pltpu.SemaphoreType.DMA] * (2 * _NBUF)
        ),
    )
    def body(kp_hbm, idx_hbm, tidx_hbm, out_hbm, idxv, tidxv, *bs):
        bufs = bs[:_NBUF]
        gsems = bs[_NBUF:2 * _NBUF]
        ssems = bs[2 * _NBUF:]
        wid = jax.lax.axis_index("s") * 2 + jax.lax.axis_index("c")
        pltpu.sync_copy(idx_hbm.at[wid], idxv)
        pltpu.sync_copy(tidx_hbm.at[wid], tidxv)
        pltpu.async_copy(kp_hbm.at[idxv.at[0]], bufs[0], gsems[0])

        @pl.loop(0, _NWIN, step=_NBUF)
        def _(w0):
            for j in range(_NBUF):
                w = w0 + j
                b, bn = j, (j + 1) % _NBUF
                pltpu.make_async_copy(kp_hbm.at[idxv.at[w]],
                                      bufs[b], gsems[b]).wait()
                pltpu.async_copy(bufs[b], out_hbm.at[tidxv.at[w]], ssems[b])

                @pl.when(w + 1 < _NWIN)
                def _():
                    @pl.when(w >= _NBUF - 1)
                    def _():
                        pltpu.make_async_copy(
                            bufs[bn], out_hbm.at[tidxv.at[0]],
                            ssems[bn]).wait()
                    pltpu.async_copy(kp_hbm.at[idxv.at[w + 1]],
                                     bufs[bn], gsems[bn])

        for t in range(_NWIN - _NBUF, _NWIN):
            pltpu.make_async_copy(bufs[t % _NBUF], out_hbm.at[tidxv.at[0]],
                                  ssems[t % _NBUF]).wait()

    return body(kpp, idx2d, tidx2d)


def kernel(x, y):
    qt = _extract_patches_t(x, _P, _S)
    kpt = _extract_patches_t(y, _P, _S)
    q, kp = _rows_from_t(qt, kpt)

    scores, idxs = pl.pallas_call(
        _topk_body,
        grid=(_NBLK,),
        in_specs=[
            pl.BlockSpec((_QB, _D), lambda i: (i, 0)),
            pl.BlockSpec((_D, _Q), lambda i: (0, 0)),
        ],
        out_specs=[
            pl.BlockSpec((_QB, 8), lambda i: (i, 0)),
            pl.BlockSpec((_QB, 8), lambda i: (i, 0)),
        ],
        out_shape=[
            jax.ShapeDtypeStruct((_QPAD, 8), jnp.float32),
            jax.ShapeDtypeStruct((_QPAD, 8), jnp.int32),
        ],
        scratch_shapes=[
            pltpu.VMEM((_D, _Q), jnp.bfloat16),
            pltpu.VMEM((8, _Q), jnp.float32),
        ],
    )(q, kpt)

    t = jnp.arange(_TPAD, dtype=jnp.int32)
    real = t < _QPAD * _K
    idx_flat = jnp.pad(idxs[:, :_K].reshape(_QPAD * _K),
                       (0, _TPAD - _QPAD * _K))
    tidx_flat = jnp.where(real, 8 * (t // _K) + t % _K,
                          8 * (t - _QPAD * _K) + 7)

    neigh8 = _sc_gather(kp, idx_flat.reshape(_NW, _NWIN, _W),
                        tidx_flat.reshape(_NW, _NWIN, _W))

    neigh = neigh8.reshape(_QPAD, 8, _D)[:_Q, :_K, :]
    diff_patch = (neigh - q[:, None, :])[None]

    pin = pl.pallas_call(
        _pin_body,
        grid=(1,),
        in_specs=[pl.BlockSpec((8, 128), lambda i: (0, 0))],
        out_specs=pl.BlockSpec((8, 128), lambda i: (0, 0)),
        out_shape=jax.ShapeDtypeStruct((8, 128), jnp.float32),
    )(neigh8)

    score_k = scores[:_Q, :_K][None] + pin[0, 0] * 0.0
    idx_k = idxs[:_Q, :_K][None]
    return (score_k, idx_k, diff_patch)


# speedup vs baseline: 1.2372x; 1.2372x over previous
import functools

import jax
import jax.numpy as jnp
from jax.experimental import pallas as pl
from jax.experimental.pallas import tpu as pltpu
from jax.experimental.pallas import tpu_sc as plsc

_K = 5
_P = 3
_S = 1

_Q = 2116
_D = 2304
_QB = 128
_NBLK = 17
_QPAD = _QB * _NBLK
_NEG = -3.0e38


def _extract_patches_t(feat, p, s):
    xs = feat[0]
    nh = xs.shape[1] - p + 1
    win = jnp.stack([xs[:, dy:dy + nh, dx:dx + nh]
                     for dy in range(p) for dx in range(p)], axis=1)
    return win.reshape(xs.shape[0] * p * p, nh * nh)


def _sub_body(n_ref, q_ref, o_ref):
    n3 = n_ref[...].reshape(_QB, 8, _D)
    qb = q_ref[...]
    o_ref[0] = (n3 - qb[:, None, :])[:, :_K, :]


def _transpose_body(qt_ref, kt_ref, q_ref, k_ref):
    q_ref[...] = qt_ref[...].T
    k_ref[...] = kt_ref[...].T


def _rows_from_t(qt, kt):
    return pl.pallas_call(
        _transpose_body,
        grid=(_NBLK,),
        in_specs=[
            pl.BlockSpec((_D, _QB), lambda i: (0, i)),
            pl.BlockSpec((_D, _QB), lambda i: (0, i)),
        ],
        out_specs=[
            pl.BlockSpec((_QB, _D), lambda i: (i, 0)),
            pl.BlockSpec((_QB, _D), lambda i: (i, 0)),
        ],
        out_shape=[
            jax.ShapeDtypeStruct((_Q, _D), jnp.float32),
            jax.ShapeDtypeStruct((_Q, _D), jnp.float32),
        ],
    )(qt, kt)


def _topk_body(q_ref, kt_ref, sc_ref, ix_ref, kb16_ref, k2_ref):
    i = pl.program_id(0)

    @pl.when(i == 0)
    def _():
        kt = kt_ref[...]
        kb16_ref[...] = kt.astype(jnp.bfloat16)
        k2_ref[0:1, :] = jnp.sum(kt * kt, axis=0, keepdims=True)

    qb = q_ref[...]
    q2 = jnp.sum(qb * qb, axis=1)
    qk = jax.lax.dot_general(
        qb.astype(jnp.bfloat16), kb16_ref[...],
        dimension_numbers=(((1,), (0,)), ((), ())),
        preferred_element_type=jnp.float32)
    t = 2.0 * qk - k2_ref[0:1, :]
    col = jax.lax.broadcasted_iota(jnp.int32, (_QB, _Q), 1)
    for kk in range(_K):
        m = jnp.max(t, axis=1)
        a = jnp.argmax(t, axis=1).astype(jnp.int32)
        sc_ref[:, kk] = m - q2
        ix_ref[:, kk] = a
        t = jnp.where(col == a[:, None], _NEG, t)


_NW = 32
_TPW = 352
_TPAD = _NW * _TPW
_W = 8
_NWIN = _TPW // _W
_NBUF = 4


def _sc_gather(kpp, idx2d, tidx2d):
    mesh = plsc.VectorSubcoreMesh(core_axis_name="c", subcore_axis_name="s")

    @functools.partial(
        pl.kernel, mesh=mesh,
        out_type=jax.ShapeDtypeStruct((_QPAD * 8, _D), jnp.float32),
        scratch_types=(
            [pltpu.VMEM((_NWIN, _W), jnp.int32),
             pltpu.VMEM((_NWIN, _W), jnp.int32)]
            + [pltpu.VMEM((_W, _D), jnp.float32)] * _NBUF
            + [pltpu.SemaphoreType.DMA] * (2 * _NBUF)
        ),
    )
    def body(kp_hbm, idx_hbm, tidx_hbm, out_hbm, idxv, tidxv, *bs):
        bufs = bs[:_NBUF]
        gsems = bs[_NBUF:2 * _NBUF]
        ssems = bs[2 * _NBUF:]
        wid = jax.lax.axis_index("s") * 2 + jax.lax.axis_index("c")
        pltpu.sync_copy(idx_hbm.at[wid], idxv)
        pltpu.sync_copy(tidx_hbm.at[wid], tidxv)
        pltpu.async_copy(kp_hbm.at[idxv.at[0]], bufs[0], gsems[0])

        @pl.loop(0, _NWIN, step=_NBUF)
        def _(w0):
            for j in range(_NBUF):
                w = w0 + j
                b, bn = j, (j + 1) % _NBUF
                pltpu.make_async_copy(kp_hbm.at[idxv.at[w]],
                                      bufs[b], gsems[b]).wait()
                pltpu.async_copy(bufs[b], out_hbm.at[tidxv.at[w]], ssems[b])

                @pl.when(w + 1 < _NWIN)
                def _():
                    @pl.when(w >= _NBUF - 1)
                    def _():
                        pltpu.make_async_copy(
                            bufs[bn], out_hbm.at[tidxv.at[0]],
                            ssems[bn]).wait()
                    pltpu.async_copy(kp_hbm.at[idxv.at[w + 1]],
                                     bufs[bn], gsems[bn])

        for t in range(_NWIN - _NBUF, _NWIN):
            pltpu.make_async_copy(bufs[t % _NBUF], out_hbm.at[tidxv.at[0]],
                                  ssems[t % _NBUF]).wait()

    return body(kpp, idx2d, tidx2d)


def kernel(x, y):
    qt = _extract_patches_t(x, _P, _S)
    kpt = _extract_patches_t(y, _P, _S)
    q, kp = _rows_from_t(qt, kpt)

    scores, idxs = pl.pallas_call(
        _topk_body,
        grid=(_NBLK,),
        in_specs=[
            pl.BlockSpec((_QB, _D), lambda i: (i, 0)),
            pl.BlockSpec((_D, _Q), lambda i: (0, 0)),
        ],
        out_specs=[
            pl.BlockSpec((_QB, 8), lambda i: (i, 0)),
            pl.BlockSpec((_QB, 8), lambda i: (i, 0)),
        ],
        out_shape=[
            jax.ShapeDtypeStruct((_QPAD, 8), jnp.float32),
            jax.ShapeDtypeStruct((_QPAD, 8), jnp.int32),
        ],
        scratch_shapes=[
            pltpu.VMEM((_D, _Q), jnp.bfloat16),
            pltpu.VMEM((8, _Q), jnp.float32),
        ],
    )(q, kpt)

    t = jnp.arange(_TPAD, dtype=jnp.int32)
    real = t < _QPAD * _K
    idx_flat = jnp.pad(idxs[:, :_K].reshape(_QPAD * _K),
                       (0, _TPAD - _QPAD * _K))
    tidx_flat = jnp.where(real, 8 * (t // _K) + t % _K,
                          8 * (t - _QPAD * _K) + 7)

    neigh8 = _sc_gather(kp, idx_flat.reshape(_NW, _NWIN, _W),
                        tidx_flat.reshape(_NW, _NWIN, _W))

    diff_patch = pl.pallas_call(
        _sub_body,
        grid=(_NBLK,),
        in_specs=[
            pl.BlockSpec((_QB * 8, _D), lambda i: (i, 0)),
            pl.BlockSpec((_QB, _D), lambda i: (i, 0)),
        ],
        out_specs=pl.BlockSpec((1, _QB, _K, _D), lambda i: (0, i, 0, 0)),
        out_shape=jax.ShapeDtypeStruct((1, _Q, _K, _D), jnp.float32),
    )(neigh8, q)

    score_k = scores[:_Q, :_K][None]
    idx_k = idxs[:_Q, :_K][None]
    return (score_k, idx_k, diff_patch)
